# baseline (speedup 1.0000x reference)
import jax
import jax.numpy as jnp
from jax.experimental import pallas as pl
from jax.experimental.pallas import tpu as pltpu

_K_MIN = 1.0
_K_MAX = 12.0
_ALPHA = 12.0
_TOPK = 12
_SENT = -1.0e6

_GRP = 4
_N = 256
_C = 768
_H = 3072
_HC = _H // 4


def _ffn_stage(g, h, x_ref, fc1_w_ref, fc1_b_ref, fc2_w_ref, fc2_b_ref,
               k1_w_ref, k1_b_ref, k2_w_ref, k2_b_ref,
               w1_w_ref, w1_b_ref, w2_w_ref, w2_b_ref, hbuf_ref, kw_ref):
    m = _GRP * _N
    xs = x_ref[...].reshape(m, _C)
    gslot = jax.lax.rem(g, 2)

    h1c = jnp.maximum(
        jnp.dot(xs, fc1_w_ref[...], preferred_element_type=jnp.float32)
        + fc1_b_ref[...], 0.0)
    part = jnp.dot(h1c, fc2_w_ref[...], preferred_element_type=jnp.float32)

    @pl.when(h == 0)
    def _init():
        hbuf_ref[gslot] = part + fc2_b_ref[...]
        prow = jax.lax.broadcasted_iota(jnp.int32, (8, m), 0)
        pcol = jax.lax.broadcasted_iota(jnp.int32, (8, m), 1)
        selmat = jnp.where(jax.lax.rem(prow, 4) == pcol // _N, 1.0 / _N, 0.0)
        pooled = jnp.dot(selmat, xs, preferred_element_type=jnp.float32)
        t1 = jnp.maximum(
            jnp.dot(pooled, k1_w_ref[...], preferred_element_type=jnp.float32)
            + k1_b_ref[...], 0.0)
        kl = jnp.dot(t1, k2_w_ref[...],
                     preferred_element_type=jnp.float32) + k2_b_ref[...]
        kc = _K_MIN + jax.nn.sigmoid(kl) * (_K_MAX - _K_MIN)
        t2 = jnp.maximum(
            jnp.dot(pooled, w1_w_ref[...], preferred_element_type=jnp.float32)
            + w1_b_ref[...], 0.0)
        wl = jnp.dot(t2, w2_w_ref[...],
                     preferred_element_type=jnp.float32) + w2_b_ref[...]
        riota = jax.lax.broadcasted_iota(jnp.int32, (8, 128), 0)
        kw_ref[gslot] = jnp.where(riota < 4, kc, wl)

    @pl.when(h > 0)
    def _acc():
        hbuf_ref[gslot] += part


def _knn_stage(g, h, o_ref, hbuf_ref, kw_ref):
    gslot = jax.lax.rem(g + 1, 2)
    hb = hbuf_ref[gslot, pl.ds(h * _N, _N), :]
    kwrow = kw_ref[gslot, pl.ds(h, 1), :]
    wlrow = kw_ref[gslot, pl.ds(h + 4, 1), :]
    k_i = [kwrow[0, i] for i in range(3)]
    l_i = [wlrow[0, i] for i in range(3)]
    lmax = jnp.maximum(jnp.maximum(l_i[0], l_i[1]), l_i[2])
    e_i = [jnp.exp(l - lmax) for l in l_i]
    esum3 = e_i[0] + e_i[1] + e_i[2]
    w_i = [e / esum3 for e in e_i]

    gram = jnp.dot(hb, hb.T, preferred_element_type=jnp.float32)
    rows = jax.lax.broadcasted_iota(jnp.int32, (_N, _N), 0)
    cols = jax.lax.broadcasted_iota(jnp.int32, (_N, _N), 1)
    eye = (rows == cols).astype(jnp.float32)
    sq_row = jnp.sum(gram * eye, axis=0, keepdims=True)
    logits = 2.0 * gram - sq_row

    work = logits
    v0 = None
    for j in range(_TOPK):
        cur = jnp.max(work, axis=1, keepdims=True)
        if j == 0:
            v0 = cur
        sel = work >= cur
        work = jnp.where(sel, _SENT - float(j + 1), work)
    rank = jnp.where(work < _SENT + 0.5, _SENT - work, 0.0)

    e = jnp.where(rank > 0.0, jnp.exp(logits - v0), 0.0)
    esum = jnp.sum(e, axis=1, keepdims=True)
    attn = jnp.zeros((_N, _N), jnp.float32)
    for i in range(3):
        mi = jax.nn.sigmoid(_ALPHA * (k_i[i] - rank))
        num_i = e * mi
        den_i = jnp.sum(num_i, axis=1, keepdims=True) + 1e-8 * esum
        attn = attn + (w_i[i] / den_i) * num_i
    o_ref[0] = jnp.dot(attn, hb, preferred_element_type=jnp.float32)


def _body(x_ref, fc1_w_ref, fc1_b_ref, fc2_w_ref, fc2_b_ref,
          k1_w_ref, k1_b_ref, k2_w_ref, k2_b_ref,
          w1_w_ref, w1_b_ref, w2_w_ref, w2_b_ref, o_ref,
          hbuf_ref, kw_ref):
    g = pl.program_id(0)
    h = pl.program_id(1)
    args = (x_ref, fc1_w_ref, fc1_b_ref, fc2_w_ref, fc2_b_ref,
            k1_w_ref, k1_b_ref, k2_w_ref, k2_b_ref,
            w1_w_ref, w1_b_ref, w2_w_ref, w2_b_ref, hbuf_ref, kw_ref)

    @pl.when(g == 0)
    def _g0():
        _ffn_stage(g, h, *args)

    @pl.when(g == 1)
    def _g1():
        _ffn_stage(g, h, *args)
        _knn_stage(g, h, o_ref, hbuf_ref, kw_ref)

    @pl.when(g == 2)
    def _g2():
        _knn_stage(g, h, o_ref, hbuf_ref, kw_ref)


def kernel(x, fc1_w, fc1_b, fc2_w, fc2_b, k1_w, k1_b, k2_w, k2_b,
           w1_w, w1_b, w2_w, w2_b):
    B, N, C = x.shape
    H = fc1_w.shape[1]
    k2_wp = jnp.pad(k2_w, ((0, 0), (0, 128 - k2_w.shape[1])))
    k2_bp = jnp.pad(k2_b, (0, 128 - k2_b.shape[0])).reshape(1, 128)
    w2_wp = jnp.pad(w2_w, ((0, 0), (0, 128 - w2_w.shape[1])))
    w2_bp = jnp.pad(w2_b, (0, 128 - w2_b.shape[0])).reshape(1, 128)

    const = lambda shape: pl.BlockSpec(shape, lambda g, h: (0,) * len(shape))
    chunk_h = lambda g, h: jnp.where(g >= 2, 3, h)
    return pl.pallas_call(
        _body,
        grid=(3, 4),
        in_specs=[
            pl.BlockSpec((_GRP, N, C), lambda g, h: (jnp.minimum(g, 1), 0, 0)),
            pl.BlockSpec((C, _HC), lambda g, h: (0, chunk_h(g, h))),
            pl.BlockSpec((1, _HC), lambda g, h: (0, chunk_h(g, h))),
            pl.BlockSpec((_HC, C), lambda g, h: (chunk_h(g, h), 0)),
            const((1, C)),
            const((C, 128)), const((1, 128)),
            const((128, 128)), const((1, 128)),
            const((C, 128)), const((1, 128)),
            const((128, 128)), const((1, 128)),
        ],
        out_specs=pl.BlockSpec(
            (1, N, C), lambda g, h: (jnp.clip(4 * g + h - 4, 0, 7), 0, 0)),
        out_shape=jax.ShapeDtypeStruct((B, N, C), jnp.float32),
        scratch_shapes=[
            pltpu.VMEM((2, _GRP * N, C), jnp.float32),
            pltpu.VMEM((2, 8, 128), jnp.float32),
        ],
    )(x, fc1_w, fc1_b.reshape(1, H), fc2_w, fc2_b.reshape(1, C),
      k1_w, k1_b.reshape(1, 128), k2_wp, k2_bp,
      w1_w, w1_b.reshape(1, 128), w2_wp, w2_bp)


# speedup vs baseline: 1.0599x; 1.0599x over previous
import jax
import jax.numpy as jnp
from jax.experimental import pallas as pl
from jax.experimental.pallas import tpu as pltpu

_K_MIN = 1.0
_K_MAX = 12.0
_ALPHA = 12.0
_TOPK = 12
_SENT = -1.0e6

_BPB = 4


def _logits_stage(hb):
    n = hb.shape[0]
    gram = jnp.dot(hb, hb.T, preferred_element_type=jnp.float32)
    rows = jax.lax.broadcasted_iota(jnp.int32, (n, n), 0)
    cols = jax.lax.broadcasted_iota(jnp.int32, (n, n), 1)
    eye = (rows == cols).astype(jnp.float32)
    sq_row = jnp.sum(gram * eye, axis=0, keepdims=True)
    return 2.0 * gram - sq_row


def _topk_stage(logits):
    work = logits
    v0 = None
    for j in range(_TOPK):
        cur = jnp.max(work, axis=1, keepdims=True)
        if j == 0:
            v0 = cur
        sel = work >= cur
        work = jnp.where(sel, _SENT - float(j + 1), work)
    rank = jnp.where(work < _SENT + 0.5, _SENT - work, 0.0)
    return rank, v0


def _final_stage(hb, logits, rank, v0, k_i, w_i):
    e = jnp.where(rank > 0.0, jnp.exp(logits - v0), 0.0)
    esum = jnp.sum(e, axis=1, keepdims=True)
    attn = jnp.zeros(logits.shape, jnp.float32)
    for i in range(3):
        mi = jax.nn.sigmoid(_ALPHA * (k_i[i] - rank))
        num_i = e * mi
        den_i = jnp.sum(num_i, axis=1, keepdims=True) + 1e-8 * esum
        attn = attn + (w_i[i] / den_i) * num_i
    return jnp.dot(attn, hb, preferred_element_type=jnp.float32)


_F1C = 8
_F2C = 4


def _body(x_ref, fc1_w_ref, fc1_b_ref, fc2_w_ref, fc2_b_ref,
          k1_w_ref, k1_b_ref, k2_w_ref, k2_b_ref,
          w1_w_ref, w1_b_ref, w2_w_ref, w2_b_ref, o_ref,
          fc1_s, fc2_s, h1_s, sem):
    n, c = x_ref.shape[1], x_ref.shape[2]
    m = _BPB * n
    hdim = fc2_s.shape[0]
    xs = x_ref[...].reshape(m, c)

    f1w = hdim // _F1C
    f2w = hdim // _F2C
    f1_copies = [
        pltpu.make_async_copy(
            fc1_w_ref.at[:, ci * f1w:(ci + 1) * f1w],
            fc1_s.at[:, ci * f1w:(ci + 1) * f1w], sem.at[ci])
        for ci in range(_F1C)]
    f2_copies = [
        pltpu.make_async_copy(
            fc2_w_ref.at[ci * f2w:(ci + 1) * f2w, :],
            fc2_s.at[ci * f2w:(ci + 1) * f2w, :], sem.at[_F1C + ci])
        for ci in range(_F2C)]
    for cp in f1_copies + f2_copies:
        cp.start()

    prow = jax.lax.broadcasted_iota(jnp.int32, (_BPB, m), 0)
    pcol = jax.lax.broadcasted_iota(jnp.int32, (_BPB, m), 1)
    selmat = jnp.where(prow == pcol // n, 1.0 / n, 0.0)
    pooled = jnp.dot(selmat, xs, preferred_element_type=jnp.float32)

    t = jnp.maximum(
        jnp.dot(pooled, k1_w_ref[...], preferred_element_type=jnp.float32)
        + k1_b_ref[...], 0.0)
    kl = jnp.dot(t, k2_w_ref[...], preferred_element_type=jnp.float32) + k2_b_ref[...]
    kc = _K_MIN + jax.nn.sigmoid(kl) * (_K_MAX - _K_MIN)
    t2 = jnp.maximum(
        jnp.dot(pooled, w1_w_ref[...], preferred_element_type=jnp.float32)
        + w1_b_ref[...], 0.0)
    wl = jnp.dot(t2, w2_w_ref[...], preferred_element_type=jnp.float32) + w2_b_ref[...]

    k_l, w_l = [], []
    for bi in range(_BPB):
        k_l.append([kc[bi, i] for i in range(3)])
        l_i = [wl[bi, i] for i in range(3)]
        lmax = jnp.maximum(jnp.maximum(l_i[0], l_i[1]), l_i[2])
        e_i = [jnp.exp(l - lmax) for l in l_i]
        esum = e_i[0] + e_i[1] + e_i[2]
        w_l.append([e / esum for e in e_i])

    for ci in range(_F1C):
        f1_copies[ci].wait()
        h1_s[:, ci * f1w:(ci + 1) * f1w] = jnp.maximum(
            jnp.dot(xs, fc1_s[:, ci * f1w:(ci + 1) * f1w],
                    preferred_element_type=jnp.float32)
            + fc1_b_ref[0:1, ci * f1w:(ci + 1) * f1w], 0.0)
    for cp in f2_copies:
        cp.wait()

    hb = [None] * _BPB
    logits = [None] * _BPB
    ranks = [None] * _BPB
    for s in range(_BPB + 2):
        if s < _BPB:
            hb[s] = jnp.dot(h1_s[s * n:(s + 1) * n, :], fc2_s[...],
                            preferred_element_type=jnp.float32) + fc2_b_ref[...]
            logits[s] = _logits_stage(hb[s])
        if 1 <= s <= _BPB:
            ranks[s - 1] = _topk_stage(logits[s - 1])
        if s >= 2:
            bi = s - 2
            o_ref[bi] = _final_stage(hb[bi], logits[bi], ranks[bi][0],
                                     ranks[bi][1], k_l[bi], w_l[bi])


def kernel(x, fc1_w, fc1_b, fc2_w, fc2_b, k1_w, k1_b, k2_w, k2_b,
           w1_w, w1_b, w2_w, w2_b):
    B, N, C = x.shape
    H = fc1_w.shape[1]
    k2_wp = jnp.pad(k2_w, ((0, 0), (0, 128 - k2_w.shape[1])))
    k2_bp = jnp.pad(k2_b, (0, 128 - k2_b.shape[0])).reshape(1, 128)
    w2_wp = jnp.pad(w2_w, ((0, 0), (0, 128 - w2_w.shape[1])))
    w2_bp = jnp.pad(w2_b, (0, 128 - w2_b.shape[0])).reshape(1, 128)

    const = lambda shape: pl.BlockSpec(shape, lambda b: (0,) * len(shape))
    return pl.pallas_call(
        _body,
        grid=(B // _BPB,),
        in_specs=[
            pl.BlockSpec((_BPB, N, C), lambda b: (b, 0, 0)),
            pl.BlockSpec(memory_space=pl.ANY), const((1, H)),
            pl.BlockSpec(memory_space=pl.ANY), const((1, C)),
            const((C, 128)), const((1, 128)),
            const((128, 128)), const((1, 128)),
            const((C, 128)), const((1, 128)),
            const((128, 128)), const((1, 128)),
        ],
        out_specs=pl.BlockSpec((_BPB, N, C), lambda b: (b, 0, 0)),
        out_shape=jax.ShapeDtypeStruct((B, N, C), jnp.float32),
        scratch_shapes=[
            pltpu.VMEM((C, H), jnp.float32),
            pltpu.VMEM((H, C), jnp.float32),
            pltpu.VMEM((_BPB * N, H), jnp.float32),
            pltpu.SemaphoreType.DMA((_F1C + _F2C,)),
        ],
    )(x, fc1_w, fc1_b.reshape(1, H), fc2_w, fc2_b.reshape(1, C),
      k1_w, k1_b.reshape(1, 128), k2_wp, k2_bp,
      w1_w, w1_b.reshape(1, 128), w2_wp, w2_bp)


# speedup vs baseline: 1.2044x; 1.1364x over previous
import jax
import jax.numpy as jnp
from jax.experimental import pallas as pl

_K_MIN = 1.0
_K_MAX = 12.0
_ALPHA = 12.0
_TOPK = 12
_SENT = -1.0e6

_BPB = 4


def _logits_stage(hb):
    n = hb.shape[0]
    gram = jnp.dot(hb, hb.T, preferred_element_type=jnp.float32)
    rows = jax.lax.broadcasted_iota(jnp.int32, (n, n), 0)
    cols = jax.lax.broadcasted_iota(jnp.int32, (n, n), 1)
    eye = (rows == cols).astype(jnp.float32)
    sq_row = jnp.sum(gram * eye, axis=0, keepdims=True)
    return 2.0 * gram - sq_row


def _topk_stage(logits):
    work = logits
    v0 = None
    for j in range(_TOPK):
        cur = jnp.max(work, axis=1, keepdims=True)
        if j == 0:
            v0 = cur
        sel = work >= cur
        work = jnp.where(sel, _SENT - float(j + 1), work)
    rank = jnp.where(work < _SENT + 0.5, _SENT - work, 0.0)
    return rank, v0


def _final_stage(hb, logits, rank, v0, k_i, w_i):
    e = jnp.where(rank > 0.0, jnp.exp(logits - v0), 0.0)
    esum = jnp.sum(e, axis=1, keepdims=True)
    attn = jnp.zeros(logits.shape, jnp.float32)
    for i in range(3):
        mi = jax.nn.sigmoid(_ALPHA * (k_i[i] - rank))
        num_i = e * mi
        den_i = jnp.sum(num_i, axis=1, keepdims=True) + 1e-8 * esum
        attn = attn + (w_i[i] / den_i) * num_i
    return jnp.dot(attn, hb, preferred_element_type=jnp.float32)


def _body(x_ref, fc1_w_ref, fc1_b_ref, fc2_w_ref, fc2_b_ref,
          k1_w_ref, k1_b_ref, k2_w_ref, k2_b_ref,
          w1_w_ref, w1_b_ref, w2_w_ref, w2_b_ref, o_ref):
    n, c = x_ref.shape[1], x_ref.shape[2]
    m = _BPB * n
    xs = x_ref[...].reshape(m, c)

    prow = jax.lax.broadcasted_iota(jnp.int32, (_BPB, m), 0)
    pcol = jax.lax.broadcasted_iota(jnp.int32, (_BPB, m), 1)
    selmat = jnp.where(prow == pcol // n, 1.0 / n, 0.0)
    pooled = jnp.dot(selmat, xs, preferred_element_type=jnp.float32)

    t = jnp.maximum(
        jnp.dot(pooled, k1_w_ref[...], preferred_element_type=jnp.float32)
        + k1_b_ref[...], 0.0)
    kl = jnp.dot(t, k2_w_ref[...], preferred_element_type=jnp.float32) + k2_b_ref[...]
    kc = _K_MIN + jax.nn.sigmoid(kl) * (_K_MAX - _K_MIN)
    t2 = jnp.maximum(
        jnp.dot(pooled, w1_w_ref[...], preferred_element_type=jnp.float32)
        + w1_b_ref[...], 0.0)
    wl = jnp.dot(t2, w2_w_ref[...], preferred_element_type=jnp.float32) + w2_b_ref[...]

    k_l, w_l = [], []
    for bi in range(_BPB):
        k_l.append([kc[bi, i] for i in range(3)])
        l_i = [wl[bi, i] for i in range(3)]
        lmax = jnp.maximum(jnp.maximum(l_i[0], l_i[1]), l_i[2])
        e_i = [jnp.exp(l - lmax) for l in l_i]
        esum = e_i[0] + e_i[1] + e_i[2]
        w_l.append([e / esum for e in e_i])

    h1 = jnp.maximum(
        jnp.dot(xs, fc1_w_ref[...], preferred_element_type=jnp.float32)
        + fc1_b_ref[...], 0.0)

    hb = [None] * _BPB
    logits = [None] * _BPB
    ranks = [None] * _BPB
    for s in range(_BPB + 2):
        if s < _BPB:
            hb[s] = jnp.dot(h1[s * n:(s + 1) * n], fc2_w_ref[...],
                            preferred_element_type=jnp.float32) + fc2_b_ref[...]
            logits[s] = _logits_stage(hb[s])
        if 1 <= s <= _BPB:
            ranks[s - 1] = _topk_stage(logits[s - 1])
        if s >= 2:
            bi = s - 2
            o_ref[bi] = _final_stage(hb[bi], logits[bi], ranks[bi][0],
                                     ranks[bi][1], k_l[bi], w_l[bi])


def kernel(x, fc1_w, fc1_b, fc2_w, fc2_b, k1_w, k1_b, k2_w, k2_b,
           w1_w, w1_b, w2_w, w2_b):
    B, N, C = x.shape
    H = fc1_w.shape[1]
    k2_wp = jnp.pad(k2_w, ((0, 0), (0, 128 - k2_w.shape[1])))
    k2_bp = jnp.pad(k2_b, (0, 128 - k2_b.shape[0])).reshape(1, 128)
    w2_wp = jnp.pad(w2_w, ((0, 0), (0, 128 - w2_w.shape[1])))
    w2_bp = jnp.pad(w2_b, (0, 128 - w2_b.shape[0])).reshape(1, 128)

    const = lambda shape: pl.BlockSpec(shape, lambda b: (0,) * len(shape))
    return pl.pallas_call(
        _body,
        grid=(B // _BPB,),
        in_specs=[
            pl.BlockSpec((_BPB, N, C), lambda b: (b, 0, 0)),
            const((C, H)), const((1, H)),
            const((H, C)), const((1, C)),
            const((C, 128)), const((1, 128)),
            const((128, 128)), const((1, 128)),
            const((C, 128)), const((1, 128)),
            const((128, 128)), const((1, 128)),
        ],
        out_specs=pl.BlockSpec((_BPB, N, C), lambda b: (b, 0, 0)),
        out_shape=jax.ShapeDtypeStruct((B, N, C), jnp.float32),
    )(x, fc1_w, fc1_b.reshape(1, H), fc2_w, fc2_b.reshape(1, C),
      k1_w, k1_b.reshape(1, 128), k2_wp, k2_bp,
      w1_w, w1_b.reshape(1, 128), w2_wp, w2_bp)
